# Initial kernel scaffold; baseline (speedup 1.0000x reference)
import jax
import jax.numpy as jnp
from jax.experimental import pallas as pl


def kernel(x, edge_index, W1, b1, W2, b2, W3, b3, W4, b4):
    raise NotImplementedError("write your pallas kernel here")


# speedup vs baseline: 5.1199x; 5.1199x over previous
import functools

import jax
import jax.numpy as jnp
from jax import lax
from jax.experimental import pallas as pl
from jax.experimental.pallas import tpu as pltpu
from jax.experimental.pallas import tpu_sc as plsc

N = 10000
E = 320000
D = 128

NC = 2
NS = 16
NW = NC * NS
EPW = E // NW
CHUNK = 80
NCHUNK = EPW // CHUNK
ROWS_MAIN = 624
ROWS_TAIL = N - NS * ROWS_MAIN


def _sc_segment_sum(h, src, dst, zeros):
  mesh = plsc.VectorSubcoreMesh(core_axis_name="c", subcore_axis_name="s")

  @functools.partial(
      pl.kernel,
      out_type=jax.ShapeDtypeStruct((NC, N, D), jnp.float32),
      mesh=mesh,
      scratch_types=[
          pltpu.VMEM((CHUNK,), jnp.int32),
          pltpu.VMEM((CHUNK,), jnp.int32),
          pltpu.VMEM((CHUNK, D), jnp.float32),
          pltpu.VMEM_SHARED((N, D), jnp.float32),
          pltpu.SemaphoreType.DMA,
      ],
  )
  def k(h_hbm, src_hbm, dst_hbm, zero_hbm, out_hbm, sidx, didx, rows, agg, sem):
    cid = lax.axis_index("c")
    sid = lax.axis_index("s")
    wid = cid * NS + sid

    r0 = sid * ROWS_MAIN
    pltpu.sync_copy(zero_hbm.at[pl.ds(r0, ROWS_MAIN)],
                    agg.at[pl.ds(r0, ROWS_MAIN)])

    @pl.when(sid == 0)
    def _():
      pltpu.sync_copy(zero_hbm.at[pl.ds(NS * ROWS_MAIN, ROWS_TAIL)],
                      agg.at[pl.ds(NS * ROWS_MAIN, ROWS_TAIL)])

    plsc.subcore_barrier()

    ebase = wid * EPW

    def body(c, carry):
      eo = ebase + c * CHUNK
      pltpu.sync_copy(src_hbm.at[pl.ds(eo, CHUNK)], sidx)
      pltpu.sync_copy(dst_hbm.at[pl.ds(eo, CHUNK)], didx)
      pltpu.async_copy(h_hbm.at[sidx], rows, sem).wait()
      pltpu.sync_copy(rows, agg.at[didx], add=True)
      return carry

    lax.fori_loop(0, NCHUNK, body, 0)
    plsc.subcore_barrier()

    pltpu.sync_copy(agg.at[pl.ds(r0, ROWS_MAIN)],
                    out_hbm.at[cid, pl.ds(r0, ROWS_MAIN)])

    @pl.when(sid == 0)
    def _():
      pltpu.sync_copy(agg.at[pl.ds(NS * ROWS_MAIN, ROWS_TAIL)],
                      out_hbm.at[cid, pl.ds(NS * ROWS_MAIN, ROWS_TAIL)])

  return k(h, src, dst, zeros)


def _tc_mlp(h, parts, Wa, ba, Wb, bb, relu_out):
  BLK = 2000
  Co = Wb.shape[1]

  def body(h_ref, p0_ref, p1_ref, wa_ref, ba_ref, wb_ref, bb_ref, o_ref):
    z = h_ref[...] + p0_ref[0] + p1_ref[0]
    t = lax.dot_general(z, wa_ref[...], (((1,), (0,)), ((), ())),
                        preferred_element_type=jnp.float32)
    t = jnp.maximum(t + ba_ref[...], 0.0)
    o = lax.dot_general(t, wb_ref[...], (((1,), (0,)), ((), ())),
                        preferred_element_type=jnp.float32)
    o = o + bb_ref[...]
    if relu_out:
      o = jnp.maximum(o, 0.0)
    o_ref[...] = o

  return pl.pallas_call(
      body,
      grid=(N // BLK,),
      in_specs=[
          pl.BlockSpec((BLK, D), lambda i: (i, 0)),
          pl.BlockSpec((1, BLK, D), lambda i: (0, i, 0)),
          pl.BlockSpec((1, BLK, D), lambda i: (1, i, 0)),
          pl.BlockSpec((D, Wa.shape[1]), lambda i: (0, 0)),
          pl.BlockSpec((1, Wa.shape[1]), lambda i: (0, 0)),
          pl.BlockSpec((Wa.shape[1], Co), lambda i: (0, 0)),
          pl.BlockSpec((1, Co), lambda i: (0, 0)),
      ],
      out_specs=pl.BlockSpec((BLK, Co), lambda i: (i, 0)),
      out_shape=jax.ShapeDtypeStruct((N, Co), jnp.float32),
  )(h, parts, parts, Wa, ba.reshape(1, -1), Wb, bb.reshape(1, -1))


def kernel(x, edge_index, W1, b1, W2, b2, W3, b3, W4, b4):
  src = edge_index[0]
  dst = edge_index[1]
  zeros = jnp.zeros((N, D), jnp.float32)
  p1 = _sc_segment_sum(x, src, dst, zeros)
  h1 = _tc_mlp(x, p1, W1, b1, W2, b2, relu_out=True)
  p2 = _sc_segment_sum(h1, src, dst, zeros)
  out = _tc_mlp(h1, p2, W3, b3, W4, b4, relu_out=False)
  return out


# speedup vs baseline: 11.5404x; 2.2540x over previous
import functools

import jax
import jax.numpy as jnp
from jax import lax
from jax.experimental import pallas as pl
from jax.experimental.pallas import tpu as pltpu
from jax.experimental.pallas import tpu_sc as plsc

N = 10000
E = 320000
D = 128

NC = 2
NS = 16
NW = NC * NS
EPW = E // NW
CHUNK = 80
NCHUNK = EPW // CHUNK
ROWS_MAIN = 624
ROWS_TAIL = N - NS * ROWS_MAIN


def _sc_segment_sum(h, src, dst3, zeros):
  mesh = plsc.VectorSubcoreMesh(core_axis_name="c", subcore_axis_name="s")

  @functools.partial(
      pl.kernel,
      out_type=jax.ShapeDtypeStruct((NC, N, D), jnp.float32),
      mesh=mesh,
      scratch_types=[
          pltpu.VMEM((EPW,), jnp.int32),
          pltpu.VMEM((NCHUNK, CHUNK), jnp.int32),
          pltpu.VMEM((CHUNK, D), jnp.float32),
          pltpu.VMEM((CHUNK, D), jnp.float32),
          pltpu.VMEM_SHARED((N, D), jnp.float32),
          pltpu.SemaphoreType.DMA,
          pltpu.SemaphoreType.DMA,
      ],
  )
  def k(h_hbm, src_hbm, dst_hbm, zero_hbm, out_hbm,
        sidx, didx, rows0, rows1, agg, sem0, sem1):
    cid = lax.axis_index("c")
    sid = lax.axis_index("s")
    wid = cid * NS + sid

    r0 = sid * ROWS_MAIN
    pltpu.sync_copy(zero_hbm.at[pl.ds(r0, ROWS_MAIN)],
                    agg.at[pl.ds(r0, ROWS_MAIN)])

    @pl.when(sid == 0)
    def _():
      pltpu.sync_copy(zero_hbm.at[pl.ds(NS * ROWS_MAIN, ROWS_TAIL)],
                      agg.at[pl.ds(NS * ROWS_MAIN, ROWS_TAIL)])

    pltpu.sync_copy(src_hbm.at[pl.ds(wid * EPW, EPW)], sidx)
    pltpu.sync_copy(dst_hbm.at[wid], didx)
    plsc.subcore_barrier()

    def sidx_of(c):
      return sidx.at[pl.ds(c * CHUNK, CHUNK)]

    pltpu.async_copy(h_hbm.at[sidx_of(0)], rows0, sem0)

    def body(i, carry):
      a = 2 * i
      pltpu.async_copy(h_hbm.at[sidx_of(a + 1)], rows1, sem1)
      pltpu.make_async_copy(h_hbm.at[sidx_of(a)], rows0, sem0).wait()
      pltpu.sync_copy(rows0, agg.at[didx.at[a]], add=True)
      pltpu.async_copy(h_hbm.at[sidx_of(a + 2)], rows0, sem0)
      pltpu.make_async_copy(h_hbm.at[sidx_of(a + 1)], rows1, sem1).wait()
      pltpu.sync_copy(rows1, agg.at[didx.at[a + 1]], add=True)
      return carry

    lax.fori_loop(0, (NCHUNK - 1) // 2, body, 0)
    pltpu.make_async_copy(h_hbm.at[sidx_of(NCHUNK - 1)], rows0, sem0).wait()
    pltpu.sync_copy(rows0, agg.at[didx.at[NCHUNK - 1]], add=True)

    plsc.subcore_barrier()

    pltpu.sync_copy(agg.at[pl.ds(r0, ROWS_MAIN)],
                    out_hbm.at[cid, pl.ds(r0, ROWS_MAIN)])

    @pl.when(sid == 0)
    def _():
      pltpu.sync_copy(agg.at[pl.ds(NS * ROWS_MAIN, ROWS_TAIL)],
                      out_hbm.at[cid, pl.ds(NS * ROWS_MAIN, ROWS_TAIL)])

  return k(h, src, dst3, zeros)


def _tc_mlp(h, parts, Wa, ba, Wb, bb, relu_out):
  BLK = 2000
  Co = Wb.shape[1]

  def body(h_ref, p0_ref, p1_ref, wa_ref, ba_ref, wb_ref, bb_ref, o_ref):
    z = h_ref[...] + p0_ref[0] + p1_ref[0]
    t = lax.dot_general(z, wa_ref[...], (((1,), (0,)), ((), ())),
                        preferred_element_type=jnp.float32)
    t = jnp.maximum(t + ba_ref[...], 0.0)
    o = lax.dot_general(t, wb_ref[...], (((1,), (0,)), ((), ())),
                        preferred_element_type=jnp.float32)
    o = o + bb_ref[...]
    if relu_out:
      o = jnp.maximum(o, 0.0)
    o_ref[...] = o

  return pl.pallas_call(
      body,
      grid=(N // BLK,),
      in_specs=[
          pl.BlockSpec((BLK, D), lambda i: (i, 0)),
          pl.BlockSpec((1, BLK, D), lambda i: (0, i, 0)),
          pl.BlockSpec((1, BLK, D), lambda i: (1, i, 0)),
          pl.BlockSpec((D, Wa.shape[1]), lambda i: (0, 0)),
          pl.BlockSpec((1, Wa.shape[1]), lambda i: (0, 0)),
          pl.BlockSpec((Wa.shape[1], Co), lambda i: (0, 0)),
          pl.BlockSpec((1, Co), lambda i: (0, 0)),
      ],
      out_specs=pl.BlockSpec((BLK, Co), lambda i: (i, 0)),
      out_shape=jax.ShapeDtypeStruct((N, Co), jnp.float32),
  )(h, parts, parts, Wa, ba.reshape(1, -1), Wb, bb.reshape(1, -1))


def kernel(x, edge_index, W1, b1, W2, b2, W3, b3, W4, b4):
  src = edge_index[0]
  dst3 = edge_index[1].reshape(NW, NCHUNK, CHUNK)
  zeros = jnp.zeros((N, D), jnp.float32)
  p1 = _sc_segment_sum(x, src, dst3, zeros)
  h1 = _tc_mlp(x, p1, W1, b1, W2, b2, relu_out=True)
  p2 = _sc_segment_sum(h1, src, dst3, zeros)
  out = _tc_mlp(h1, p2, W3, b3, W4, b4, relu_out=False)
  return out
